# baseline (speedup 1.0000x reference)
import functools

import jax
import jax.numpy as jnp
from jax import lax
from jax.experimental import pallas as pl
from jax.experimental.pallas import tpu as pltpu
from jax.experimental.pallas import tpu_sc as plsc

N = 10000
NP = 10240
HID = 64
CHUNK = 128


def _deg_kernel(nc, ns, cpw):
    nw = nc * ns
    rps = NP // ns
    mesh = plsc.VectorSubcoreMesh(core_axis_name="c", subcore_axis_name="s")

    @functools.partial(
        pl.kernel, mesh=mesh,
        compiler_params=pltpu.CompilerParams(use_tc_tiling_on_sc=False),
        out_type=jax.ShapeDtypeStruct((nc, NP), jnp.float32),
        scratch_types=[
            pltpu.VMEM((cpw + _NBUF, CHUNK), jnp.int32),
            pltpu.VMEM((CHUNK,), jnp.float32),
            pltpu.VMEM_SHARED((NP,), jnp.float32),
        ],
    )
    def k(dst_hbm, ones_hbm, zeros_hbm, out_hbm, dst_v, ones_v, acc):
        c = lax.axis_index("c")
        s = lax.axis_index("s")
        w = s * nc + c
        pltpu.sync_copy(dst_hbm.at[w], dst_v)
        pltpu.sync_copy(ones_hbm, ones_v)
        pltpu.sync_copy(zeros_hbm.at[pl.ds(s * rps, rps)],
                        acc.at[pl.ds(s * rps, rps)])
        plsc.subcore_barrier()

        def body(j, carry):
            pltpu.sync_copy(ones_v, acc.at[dst_v.at[j]], add=True)
            return carry

        lax.fori_loop(0, cpw, body, 0)
        plsc.subcore_barrier()
        pltpu.sync_copy(acc.at[pl.ds(s * rps, rps)],
                        out_hbm.at[c, pl.ds(s * rps, rps)])

    return k


_NBUF = 4


def _agg_kernel(nc, ns, cpw):
    nw = nc * ns
    rps = NP // ns
    cps = cpw + _NBUF
    mesh = plsc.VectorSubcoreMesh(core_axis_name="c", subcore_axis_name="s")

    @functools.partial(
        pl.kernel, mesh=mesh,
        compiler_params=pltpu.CompilerParams(use_tc_tiling_on_sc=False),
        out_type=jax.ShapeDtypeStruct((nc, NP, HID), jnp.float32),
        scratch_types=[
            pltpu.VMEM((cps, CHUNK), jnp.int32),
            pltpu.VMEM((cps, CHUNK), jnp.int32),
            pltpu.VMEM((CHUNK, HID), jnp.float32),
            pltpu.VMEM_SHARED((NP, HID), jnp.float32),
            pltpu.VMEM_SHARED((NP, HID), jnp.float32),
        ],
    )
    def k(y_hbm, src_hbm, dst_hbm, zeros_hbm, out_hbm,
          src_v, dst_v, row, ybuf, acc):
        c = lax.axis_index("c")
        s = lax.axis_index("s")
        w = s * nc + c
        pltpu.sync_copy(src_hbm.at[w], src_v)
        pltpu.sync_copy(dst_hbm.at[w], dst_v)
        pltpu.sync_copy(y_hbm.at[pl.ds(s * rps, rps)],
                        ybuf.at[pl.ds(s * rps, rps)])
        pltpu.sync_copy(zeros_hbm.at[pl.ds(s * rps, rps)],
                        acc.at[pl.ds(s * rps, rps)])
        plsc.subcore_barrier()

        def body(j, carry):
            pltpu.sync_copy(ybuf.at[src_v.at[j]], row)
            pltpu.sync_copy(row, acc.at[dst_v.at[j]], add=True)
            return carry

        lax.fori_loop(0, cpw, body, 0)
        plsc.subcore_barrier()
        pltpu.sync_copy(acc.at[pl.ds(s * rps, rps)],
                        out_hbm.at[c, pl.ds(s * rps, rps)])

    return k


_BR = 1024


def _xw1_call(xp, W1):
    grid = (NP // _BR,)

    def body(x_ref, w_ref, out_ref):
        out_ref[...] = jnp.dot(x_ref[...], w_ref[...],
                               preferred_element_type=jnp.float32)

    return pl.pallas_call(
        body,
        grid=grid,
        in_specs=[
            pl.BlockSpec((_BR, 128), lambda i: (i, 0)),
            pl.BlockSpec((128, HID), lambda i: (0, 0)),
        ],
        out_specs=pl.BlockSpec((_BR, HID), lambda i: (i, 0)),
        out_shape=jax.ShapeDtypeStruct((NP, HID), jnp.float32),
    )(xp, W1)


def _y1_call(xw1, degT, nc):
    grid = (NP // _BR,)

    def body(xw_ref, deg_ref, y_ref, dinv_ref):
        i = pl.program_id(0)
        deg = jnp.sum(deg_ref[...], axis=1, keepdims=True) + 1.0
        rid = i * _BR + lax.broadcasted_iota(jnp.int32, (_BR, 1), 0)
        dinv = jnp.where(rid < N, lax.rsqrt(deg), 0.0)
        y_ref[...] = xw_ref[...] * dinv
        dinv_ref[...] = dinv

    return pl.pallas_call(
        body,
        grid=grid,
        in_specs=[
            pl.BlockSpec((_BR, HID), lambda i: (i, 0)),
            pl.BlockSpec((_BR, nc), lambda i: (i, 0)),
        ],
        out_specs=[
            pl.BlockSpec((_BR, HID), lambda i: (i, 0)),
            pl.BlockSpec((_BR, 1), lambda i: (i, 0)),
        ],
        out_shape=[
            jax.ShapeDtypeStruct((NP, HID), jnp.float32),
            jax.ShapeDtypeStruct((NP, 1), jnp.float32),
        ],
    )(xw1, degT)


def _y2_call(y1, dinv, agg1p, b1r, W2, nc):
    grid = (NP // _BR,)

    def body(y_ref, dinv_ref, agg_ref, b_ref, w_ref, out_ref):
        agg = agg_ref[0]
        for c in range(1, nc):
            agg = agg + agg_ref[c]
        h = jnp.maximum(dinv_ref[...] * (agg + y_ref[...]) + b_ref[...], 0.0)
        out_ref[...] = jnp.dot(h, w_ref[...],
                               preferred_element_type=jnp.float32) * dinv_ref[...]

    return pl.pallas_call(
        body,
        grid=grid,
        in_specs=[
            pl.BlockSpec((_BR, HID), lambda i: (i, 0)),
            pl.BlockSpec((_BR, 1), lambda i: (i, 0)),
            pl.BlockSpec((nc, _BR, HID), lambda i: (0, i, 0)),
            pl.BlockSpec((1, HID), lambda i: (0, 0)),
            pl.BlockSpec((HID, HID), lambda i: (0, 0)),
        ],
        out_specs=pl.BlockSpec((_BR, HID), lambda i: (i, 0)),
        out_shape=jax.ShapeDtypeStruct((NP, HID), jnp.float32),
    )(y1, dinv, agg1p, b1r, W2)


def _h2_call(y2, dinv, agg2p, b2r, nc):
    grid = (NP // _BR,)

    def body(y_ref, dinv_ref, agg_ref, b_ref, out_ref):
        agg = agg_ref[0]
        for c in range(1, nc):
            agg = agg + agg_ref[c]
        out_ref[...] = jnp.maximum(
            dinv_ref[...] * (agg + y_ref[...]) + b_ref[...], 0.0)

    return pl.pallas_call(
        body,
        grid=grid,
        in_specs=[
            pl.BlockSpec((_BR, HID), lambda i: (i, 0)),
            pl.BlockSpec((_BR, 1), lambda i: (i, 0)),
            pl.BlockSpec((nc, _BR, HID), lambda i: (0, i, 0)),
            pl.BlockSpec((1, HID), lambda i: (0, 0)),
        ],
        out_specs=pl.BlockSpec((_BR, HID), lambda i: (i, 0)),
        out_shape=jax.ShapeDtypeStruct((NP, HID), jnp.float32),
    )(y2, dinv, agg2p, b2r)


def _sim_call(h2):
    BM = 512
    gm = (N + BM - 1) // BM

    def body(a_ref, b_ref, o_ref):
        v = lax.dot_general(a_ref[...], b_ref[...],
                            (((1,), (1,)), ((), ())),
                            preferred_element_type=jnp.float32)
        o_ref[...] = 0.5 * (1.0 + jnp.tanh(0.5 * v[:, :N]))

    return pl.pallas_call(
        body,
        grid=(gm,),
        in_specs=[
            pl.BlockSpec((BM, HID), lambda i: (i, 0)),
            pl.BlockSpec((NP, HID), lambda i: (0, 0)),
        ],
        out_specs=pl.BlockSpec((BM, N), lambda i: (i, 0)),
        out_shape=jax.ShapeDtypeStruct((N, N), jnp.float32),
    )(h2, h2)


def kernel(x, edge_index, W1, b1, W2, b2):
    info = plsc.get_sparse_core_info()
    nc, ns = info.num_cores, info.num_subcores
    nw = nc * ns

    e = edge_index.shape[1]
    unit = nw * CHUNK
    ep = ((e + unit - 1) // unit) * unit
    cpw = ep // unit
    pad = ep - e

    src = edge_index[0].astype(jnp.int32)
    dst = edge_index[1].astype(jnp.int32)
    pad_i = jnp.arange(pad, dtype=jnp.int32) % (NP - N)
    src_r = jnp.concatenate([src, N + pad_i]).reshape(nw, cpw, CHUNK)
    dst_r = jnp.concatenate([dst, N + pad_i]).reshape(nw, cpw, CHUNK)
    tail_i = (jnp.arange(_NBUF * CHUNK, dtype=jnp.int32) % (NP - N)) + N
    tail = jnp.broadcast_to(tail_i.reshape(1, _NBUF, CHUNK),
                            (nw, _NBUF, CHUNK))
    src_r = jnp.concatenate([src_r, tail], axis=1)
    dst_r = jnp.concatenate([dst_r, tail], axis=1)

    ones_c = jnp.ones((CHUNK,), jnp.float32)
    zeros_n = jnp.zeros((NP,), jnp.float32)
    zeros_nh = jnp.zeros((NP, HID), jnp.float32)

    degp = _deg_kernel(nc, ns, cpw)(dst_r, ones_c, zeros_n)
    degT = degp.T

    xp = jnp.pad(x, ((0, NP - N), (0, 0)))
    b1r = b1.reshape(1, HID)
    b2r = b2.reshape(1, HID)

    xw1 = _xw1_call(xp, W1)
    y1, dinv = _y1_call(xw1, degT, nc)
    agg1p = _agg_kernel(nc, ns, cpw)(y1, src_r, dst_r, zeros_nh)
    y2 = _y2_call(y1, dinv, agg1p, b1r, W2, nc)
    agg2p = _agg_kernel(nc, ns, cpw)(y2, src_r, dst_r, zeros_nh)
    h2 = _h2_call(y2, dinv, agg2p, b2r, nc)
    return _sim_call(h2)


# speedup vs baseline: 1.0704x; 1.0704x over previous
import functools

import jax
import jax.numpy as jnp
from jax import lax
from jax.experimental import pallas as pl
from jax.experimental.pallas import tpu as pltpu
from jax.experimental.pallas import tpu_sc as plsc

N = 10000
NP = 10240
HID = 64
CHUNK = 128


def _deg_kernel(nc, ns, cpw):
    nw = nc * ns
    rps = NP // ns
    mesh = plsc.VectorSubcoreMesh(core_axis_name="c", subcore_axis_name="s")

    @functools.partial(
        pl.kernel, mesh=mesh,
        compiler_params=pltpu.CompilerParams(use_tc_tiling_on_sc=False),
        out_type=jax.ShapeDtypeStruct((nc, NP), jnp.float32),
        scratch_types=[
            pltpu.VMEM((cpw + _NBUF, CHUNK), jnp.int32),
            pltpu.VMEM((CHUNK,), jnp.float32),
            pltpu.VMEM_SHARED((NP,), jnp.float32),
        ],
    )
    def k(dst_hbm, ones_hbm, zeros_hbm, out_hbm, dst_v, ones_v, acc):
        c = lax.axis_index("c")
        s = lax.axis_index("s")
        w = s * nc + c
        pltpu.sync_copy(dst_hbm.at[w], dst_v)
        pltpu.sync_copy(ones_hbm, ones_v)
        pltpu.sync_copy(zeros_hbm.at[pl.ds(s * rps, rps)],
                        acc.at[pl.ds(s * rps, rps)])
        plsc.subcore_barrier()

        def body(j, carry):
            pltpu.sync_copy(ones_v, acc.at[dst_v.at[j]], add=True)
            return carry

        lax.fori_loop(0, cpw, body, 0)
        plsc.subcore_barrier()
        pltpu.sync_copy(acc.at[pl.ds(s * rps, rps)],
                        out_hbm.at[c, pl.ds(s * rps, rps)])

    return k


_NBUF = 4


def _agg_kernel(nc, ns, cpw):
    nw = nc * ns
    rps = NP // ns
    cps = cpw + _NBUF
    mesh = plsc.VectorSubcoreMesh(core_axis_name="c", subcore_axis_name="s")

    @functools.partial(
        pl.kernel, mesh=mesh,
        compiler_params=pltpu.CompilerParams(use_tc_tiling_on_sc=False),
        out_type=jax.ShapeDtypeStruct((nc, NP, HID), jnp.float32),
        scratch_types=[
            pltpu.VMEM((cps, CHUNK), jnp.int32),
            pltpu.VMEM((cps, CHUNK), jnp.int32),
            [pltpu.VMEM((CHUNK, HID), jnp.float32) for _ in range(_NBUF)],
            [pltpu.SemaphoreType.DMA for _ in range(_NBUF)],
            pltpu.VMEM_SHARED((NP, HID), jnp.float32),
            pltpu.VMEM_SHARED((NP, HID), jnp.float32),
        ],
    )
    def k(y_hbm, src_hbm, dst_hbm, zeros_hbm, out_hbm,
          src_v, dst_v, rows, gsems, ybuf, acc):
        c = lax.axis_index("c")
        s = lax.axis_index("s")
        w = s * nc + c
        pltpu.sync_copy(src_hbm.at[w], src_v)
        pltpu.sync_copy(dst_hbm.at[w], dst_v)
        pltpu.sync_copy(y_hbm.at[pl.ds(s * rps, rps)],
                        ybuf.at[pl.ds(s * rps, rps)])
        pltpu.sync_copy(zeros_hbm.at[pl.ds(s * rps, rps)],
                        acc.at[pl.ds(s * rps, rps)])
        plsc.subcore_barrier()

        for b in range(_NBUF):
            pltpu.async_copy(ybuf.at[src_v.at[b]], rows[b], gsems[b])

        def body(g, carry):
            for b in range(_NBUF):
                j = g + b
                pltpu.make_async_copy(ybuf.at[src_v.at[j]],
                                      rows[b], gsems[b]).wait()
                pltpu.sync_copy(rows[b], acc.at[dst_v.at[j]], add=True)
                pltpu.async_copy(ybuf.at[src_v.at[j + _NBUF]],
                                 rows[b], gsems[b])
            return carry

        lax.fori_loop(0, cpw // _NBUF, lambda g, cr: body(g * _NBUF, cr), 0)
        for b in range(_NBUF):
            pltpu.make_async_copy(ybuf.at[src_v.at[cpw + b]],
                                  rows[b], gsems[b]).wait()
        plsc.subcore_barrier()
        pltpu.sync_copy(acc.at[pl.ds(s * rps, rps)],
                        out_hbm.at[c, pl.ds(s * rps, rps)])

    return k


_BR = 1024


def _xw1_call(xp, W1):
    grid = (NP // _BR,)

    def body(x_ref, w_ref, out_ref):
        out_ref[...] = jnp.dot(x_ref[...], w_ref[...],
                               preferred_element_type=jnp.float32)

    return pl.pallas_call(
        body,
        grid=grid,
        in_specs=[
            pl.BlockSpec((_BR, 128), lambda i: (i, 0)),
            pl.BlockSpec((128, HID), lambda i: (0, 0)),
        ],
        out_specs=pl.BlockSpec((_BR, HID), lambda i: (i, 0)),
        out_shape=jax.ShapeDtypeStruct((NP, HID), jnp.float32),
    )(xp, W1)


def _y1_call(xw1, degT, nc):
    grid = (NP // _BR,)

    def body(xw_ref, deg_ref, y_ref, dinv_ref):
        i = pl.program_id(0)
        deg = jnp.sum(deg_ref[...], axis=1, keepdims=True) + 1.0
        rid = i * _BR + lax.broadcasted_iota(jnp.int32, (_BR, 1), 0)
        dinv = jnp.where(rid < N, lax.rsqrt(deg), 0.0)
        y_ref[...] = xw_ref[...] * dinv
        dinv_ref[...] = dinv

    return pl.pallas_call(
        body,
        grid=grid,
        in_specs=[
            pl.BlockSpec((_BR, HID), lambda i: (i, 0)),
            pl.BlockSpec((_BR, nc), lambda i: (i, 0)),
        ],
        out_specs=[
            pl.BlockSpec((_BR, HID), lambda i: (i, 0)),
            pl.BlockSpec((_BR, 1), lambda i: (i, 0)),
        ],
        out_shape=[
            jax.ShapeDtypeStruct((NP, HID), jnp.float32),
            jax.ShapeDtypeStruct((NP, 1), jnp.float32),
        ],
    )(xw1, degT)


def _y2_call(y1, dinv, agg1p, b1r, W2, nc):
    grid = (NP // _BR,)

    def body(y_ref, dinv_ref, agg_ref, b_ref, w_ref, out_ref):
        agg = agg_ref[0]
        for c in range(1, nc):
            agg = agg + agg_ref[c]
        h = jnp.maximum(dinv_ref[...] * (agg + y_ref[...]) + b_ref[...], 0.0)
        out_ref[...] = jnp.dot(h, w_ref[...],
                               preferred_element_type=jnp.float32) * dinv_ref[...]

    return pl.pallas_call(
        body,
        grid=grid,
        in_specs=[
            pl.BlockSpec((_BR, HID), lambda i: (i, 0)),
            pl.BlockSpec((_BR, 1), lambda i: (i, 0)),
            pl.BlockSpec((nc, _BR, HID), lambda i: (0, i, 0)),
            pl.BlockSpec((1, HID), lambda i: (0, 0)),
            pl.BlockSpec((HID, HID), lambda i: (0, 0)),
        ],
        out_specs=pl.BlockSpec((_BR, HID), lambda i: (i, 0)),
        out_shape=jax.ShapeDtypeStruct((NP, HID), jnp.float32),
    )(y1, dinv, agg1p, b1r, W2)


def _h2_call(y2, dinv, agg2p, b2r, nc):
    grid = (NP // _BR,)

    def body(y_ref, dinv_ref, agg_ref, b_ref, out_ref):
        agg = agg_ref[0]
        for c in range(1, nc):
            agg = agg + agg_ref[c]
        out_ref[...] = jnp.maximum(
            dinv_ref[...] * (agg + y_ref[...]) + b_ref[...], 0.0)

    return pl.pallas_call(
        body,
        grid=grid,
        in_specs=[
            pl.BlockSpec((_BR, HID), lambda i: (i, 0)),
            pl.BlockSpec((_BR, 1), lambda i: (i, 0)),
            pl.BlockSpec((nc, _BR, HID), lambda i: (0, i, 0)),
            pl.BlockSpec((1, HID), lambda i: (0, 0)),
        ],
        out_specs=pl.BlockSpec((_BR, HID), lambda i: (i, 0)),
        out_shape=jax.ShapeDtypeStruct((NP, HID), jnp.float32),
    )(y2, dinv, agg2p, b2r)


def _sim_call(h2):
    BM = 512
    gm = (N + BM - 1) // BM

    def body(a_ref, b_ref, o_ref):
        v = lax.dot_general(a_ref[...], b_ref[...],
                            (((1,), (1,)), ((), ())),
                            preferred_element_type=jnp.float32)
        o_ref[...] = 0.5 * (1.0 + jnp.tanh(0.5 * v[:, :N]))

    return pl.pallas_call(
        body,
        grid=(gm,),
        in_specs=[
            pl.BlockSpec((BM, HID), lambda i: (i, 0)),
            pl.BlockSpec((NP, HID), lambda i: (0, 0)),
        ],
        out_specs=pl.BlockSpec((BM, N), lambda i: (i, 0)),
        out_shape=jax.ShapeDtypeStruct((N, N), jnp.float32),
    )(h2, h2)


def kernel(x, edge_index, W1, b1, W2, b2):
    info = plsc.get_sparse_core_info()
    nc, ns = info.num_cores, info.num_subcores
    nw = nc * ns

    e = edge_index.shape[1]
    unit = nw * CHUNK
    ep = ((e + unit - 1) // unit) * unit
    cpw = ep // unit
    pad = ep - e

    src = edge_index[0].astype(jnp.int32)
    dst = edge_index[1].astype(jnp.int32)
    pad_i = jnp.arange(pad, dtype=jnp.int32) % (NP - N)
    src_r = jnp.concatenate([src, N + pad_i]).reshape(nw, cpw, CHUNK)
    dst_r = jnp.concatenate([dst, N + pad_i]).reshape(nw, cpw, CHUNK)
    tail_i = (jnp.arange(_NBUF * CHUNK, dtype=jnp.int32) % (NP - N)) + N
    tail = jnp.broadcast_to(tail_i.reshape(1, _NBUF, CHUNK),
                            (nw, _NBUF, CHUNK))
    src_r = jnp.concatenate([src_r, tail], axis=1)
    dst_r = jnp.concatenate([dst_r, tail], axis=1)

    ones_c = jnp.ones((CHUNK,), jnp.float32)
    zeros_n = jnp.zeros((NP,), jnp.float32)
    zeros_nh = jnp.zeros((NP, HID), jnp.float32)

    degp = _deg_kernel(nc, ns, cpw)(dst_r, ones_c, zeros_n)
    degT = degp.T

    xp = jnp.pad(x, ((0, NP - N), (0, 0)))
    b1r = b1.reshape(1, HID)
    b2r = b2.reshape(1, HID)

    xw1 = _xw1_call(xp, W1)
    y1, dinv = _y1_call(xw1, degT, nc)
    agg1p = _agg_kernel(nc, ns, cpw)(y1, src_r, dst_r, zeros_nh)
    y2 = _y2_call(y1, dinv, agg1p, b1r, W2, nc)
    agg2p = _agg_kernel(nc, ns, cpw)(y2, src_r, dst_r, zeros_nh)
    h2 = _h2_call(y2, dinv, agg2p, b2r, nc)
    return _sim_call(h2)


# speedup vs baseline: 1.1555x; 1.0795x over previous
import functools

import jax
import jax.numpy as jnp
from jax import lax
from jax.experimental import pallas as pl
from jax.experimental.pallas import tpu as pltpu
from jax.experimental.pallas import tpu_sc as plsc

N = 10000
NP = 10240
HID = 64
CHUNK = 128


def _deg_kernel(nc, ns, cpw):
    nw = nc * ns
    rps = NP // ns
    mesh = plsc.VectorSubcoreMesh(core_axis_name="c", subcore_axis_name="s")

    @functools.partial(
        pl.kernel, mesh=mesh,
        compiler_params=pltpu.CompilerParams(use_tc_tiling_on_sc=False),
        out_type=jax.ShapeDtypeStruct((nc, NP), jnp.float32),
        scratch_types=[
            pltpu.VMEM((cpw + _NBUF, CHUNK), jnp.int32),
            pltpu.VMEM((CHUNK,), jnp.float32),
            pltpu.VMEM_SHARED((NP,), jnp.float32),
        ],
    )
    def k(dst_hbm, ones_hbm, zeros_hbm, out_hbm, dst_v, ones_v, acc):
        c = lax.axis_index("c")
        s = lax.axis_index("s")
        w = s * nc + c
        pltpu.sync_copy(dst_hbm.at[w], dst_v)
        pltpu.sync_copy(ones_hbm, ones_v)
        pltpu.sync_copy(zeros_hbm.at[pl.ds(s * rps, rps)],
                        acc.at[pl.ds(s * rps, rps)])
        plsc.subcore_barrier()

        def body(j, carry):
            pltpu.sync_copy(ones_v, acc.at[dst_v.at[j]], add=True)
            return carry

        lax.fori_loop(0, cpw, body, 0)
        plsc.subcore_barrier()
        pltpu.sync_copy(acc.at[pl.ds(s * rps, rps)],
                        out_hbm.at[c, pl.ds(s * rps, rps)])

    return k


_NBUF = 4


def _agg_kernel(nc, ns, cpw):
    nw = nc * ns
    rps = NP // ns
    cps = cpw + _NBUF
    mesh = plsc.VectorSubcoreMesh(core_axis_name="c", subcore_axis_name="s")

    @functools.partial(
        pl.kernel, mesh=mesh,
        compiler_params=pltpu.CompilerParams(use_tc_tiling_on_sc=False),
        out_type=jax.ShapeDtypeStruct((nc, NP, HID), jnp.float32),
        scratch_types=[
            pltpu.VMEM((cps, CHUNK), jnp.int32),
            pltpu.VMEM((cps, CHUNK), jnp.int32),
            [pltpu.VMEM((CHUNK, HID), jnp.float32) for _ in range(_NBUF)],
            [pltpu.SemaphoreType.DMA for _ in range(_NBUF)],
            pltpu.VMEM_SHARED((NP, HID), jnp.float32),
        ],
    )
    def k(y_hbm, src_hbm, dst_hbm, zeros_hbm, out_hbm,
          src_v, dst_v, rows, gsems, acc):
        c = lax.axis_index("c")
        s = lax.axis_index("s")
        w = s * nc + c
        pltpu.sync_copy(src_hbm.at[w], src_v)
        pltpu.sync_copy(dst_hbm.at[w], dst_v)
        pltpu.sync_copy(zeros_hbm.at[pl.ds(s * rps, rps)],
                        acc.at[pl.ds(s * rps, rps)])
        plsc.subcore_barrier()

        for b in range(_NBUF):
            pltpu.async_copy(y_hbm.at[src_v.at[b]], rows[b], gsems[b])

        def body(g, carry):
            for b in range(_NBUF):
                j = g + b
                pltpu.make_async_copy(y_hbm.at[src_v.at[j]],
                                      rows[b], gsems[b]).wait()
                pltpu.sync_copy(rows[b], acc.at[dst_v.at[j]], add=True)
                pltpu.async_copy(y_hbm.at[src_v.at[j + _NBUF]],
                                 rows[b], gsems[b])
            return carry

        lax.fori_loop(0, cpw // _NBUF, lambda g, cr: body(g * _NBUF, cr), 0)
        for b in range(_NBUF):
            pltpu.make_async_copy(y_hbm.at[src_v.at[cpw + b]],
                                  rows[b], gsems[b]).wait()
        plsc.subcore_barrier()
        pltpu.sync_copy(acc.at[pl.ds(s * rps, rps)],
                        out_hbm.at[c, pl.ds(s * rps, rps)])

    return k


_BR = 1024


def _xw1_call(xp, W1):
    grid = (NP // _BR,)

    def body(x_ref, w_ref, out_ref):
        out_ref[...] = jnp.dot(x_ref[...], w_ref[...],
                               preferred_element_type=jnp.float32)

    return pl.pallas_call(
        body,
        grid=grid,
        in_specs=[
            pl.BlockSpec((_BR, 128), lambda i: (i, 0)),
            pl.BlockSpec((128, HID), lambda i: (0, 0)),
        ],
        out_specs=pl.BlockSpec((_BR, HID), lambda i: (i, 0)),
        out_shape=jax.ShapeDtypeStruct((NP, HID), jnp.float32),
    )(xp, W1)


def _y1_call(xw1, degT, nc):
    grid = (NP // _BR,)

    def body(xw_ref, deg_ref, y_ref, dinv_ref):
        i = pl.program_id(0)
        deg = jnp.sum(deg_ref[...], axis=1, keepdims=True) + 1.0
        rid = i * _BR + lax.broadcasted_iota(jnp.int32, (_BR, 1), 0)
        dinv = jnp.where(rid < N, lax.rsqrt(deg), 0.0)
        y_ref[...] = xw_ref[...] * dinv
        dinv_ref[...] = dinv

    return pl.pallas_call(
        body,
        grid=grid,
        in_specs=[
            pl.BlockSpec((_BR, HID), lambda i: (i, 0)),
            pl.BlockSpec((_BR, nc), lambda i: (i, 0)),
        ],
        out_specs=[
            pl.BlockSpec((_BR, HID), lambda i: (i, 0)),
            pl.BlockSpec((_BR, 1), lambda i: (i, 0)),
        ],
        out_shape=[
            jax.ShapeDtypeStruct((NP, HID), jnp.float32),
            jax.ShapeDtypeStruct((NP, 1), jnp.float32),
        ],
    )(xw1, degT)


def _y2_call(y1, dinv, agg1p, b1r, W2, nc):
    grid = (NP // _BR,)

    def body(y_ref, dinv_ref, agg_ref, b_ref, w_ref, out_ref):
        agg = agg_ref[0]
        for c in range(1, nc):
            agg = agg + agg_ref[c]
        h = jnp.maximum(dinv_ref[...] * (agg + y_ref[...]) + b_ref[...], 0.0)
        out_ref[...] = jnp.dot(h, w_ref[...],
                               preferred_element_type=jnp.float32) * dinv_ref[...]

    return pl.pallas_call(
        body,
        grid=grid,
        in_specs=[
            pl.BlockSpec((_BR, HID), lambda i: (i, 0)),
            pl.BlockSpec((_BR, 1), lambda i: (i, 0)),
            pl.BlockSpec((nc, _BR, HID), lambda i: (0, i, 0)),
            pl.BlockSpec((1, HID), lambda i: (0, 0)),
            pl.BlockSpec((HID, HID), lambda i: (0, 0)),
        ],
        out_specs=pl.BlockSpec((_BR, HID), lambda i: (i, 0)),
        out_shape=jax.ShapeDtypeStruct((NP, HID), jnp.float32),
    )(y1, dinv, agg1p, b1r, W2)


def _h2_call(y2, dinv, agg2p, b2r, nc):
    grid = (NP // _BR,)

    def body(y_ref, dinv_ref, agg_ref, b_ref, out_ref):
        agg = agg_ref[0]
        for c in range(1, nc):
            agg = agg + agg_ref[c]
        out_ref[...] = jnp.maximum(
            dinv_ref[...] * (agg + y_ref[...]) + b_ref[...], 0.0)

    return pl.pallas_call(
        body,
        grid=grid,
        in_specs=[
            pl.BlockSpec((_BR, HID), lambda i: (i, 0)),
            pl.BlockSpec((_BR, 1), lambda i: (i, 0)),
            pl.BlockSpec((nc, _BR, HID), lambda i: (0, i, 0)),
            pl.BlockSpec((1, HID), lambda i: (0, 0)),
        ],
        out_specs=pl.BlockSpec((_BR, HID), lambda i: (i, 0)),
        out_shape=jax.ShapeDtypeStruct((NP, HID), jnp.float32),
    )(y2, dinv, agg2p, b2r)


def _sim_call(h2):
    BM = 512
    gm = (N + BM - 1) // BM

    def body(a_ref, b_ref, o_ref):
        v = lax.dot_general(a_ref[...], b_ref[...],
                            (((1,), (1,)), ((), ())),
                            preferred_element_type=jnp.float32)
        o_ref[...] = 0.5 * (1.0 + jnp.tanh(0.5 * v[:, :N]))

    return pl.pallas_call(
        body,
        grid=(gm,),
        in_specs=[
            pl.BlockSpec((BM, HID), lambda i: (i, 0)),
            pl.BlockSpec((NP, HID), lambda i: (0, 0)),
        ],
        out_specs=pl.BlockSpec((BM, N), lambda i: (i, 0)),
        out_shape=jax.ShapeDtypeStruct((N, N), jnp.float32),
    )(h2, h2)


def kernel(x, edge_index, W1, b1, W2, b2):
    info = plsc.get_sparse_core_info()
    nc, ns = info.num_cores, info.num_subcores
    nw = nc * ns

    e = edge_index.shape[1]
    unit = nw * CHUNK
    ep = ((e + unit - 1) // unit) * unit
    cpw = ep // unit
    pad = ep - e

    src = edge_index[0].astype(jnp.int32)
    dst = edge_index[1].astype(jnp.int32)
    pad_i = jnp.arange(pad, dtype=jnp.int32) % (NP - N)
    src_r = jnp.concatenate([src, N + pad_i]).reshape(nw, cpw, CHUNK)
    dst_r = jnp.concatenate([dst, N + pad_i]).reshape(nw, cpw, CHUNK)
    tail_i = (jnp.arange(_NBUF * CHUNK, dtype=jnp.int32) % (NP - N)) + N
    tail = jnp.broadcast_to(tail_i.reshape(1, _NBUF, CHUNK),
                            (nw, _NBUF, CHUNK))
    src_r = jnp.concatenate([src_r, tail], axis=1)
    dst_r = jnp.concatenate([dst_r, tail], axis=1)

    ones_c = jnp.ones((CHUNK,), jnp.float32)
    zeros_n = jnp.zeros((NP,), jnp.float32)
    zeros_nh = jnp.zeros((NP, HID), jnp.float32)

    degp = _deg_kernel(nc, ns, cpw)(dst_r, ones_c, zeros_n)
    degT = degp.T

    xp = jnp.pad(x, ((0, NP - N), (0, 0)))
    b1r = b1.reshape(1, HID)
    b2r = b2.reshape(1, HID)

    xw1 = _xw1_call(xp, W1)
    y1, dinv = _y1_call(xw1, degT, nc)
    agg1p = _agg_kernel(nc, ns, cpw)(y1, src_r, dst_r, zeros_nh)
    y2 = _y2_call(y1, dinv, agg1p, b1r, W2, nc)
    agg2p = _agg_kernel(nc, ns, cpw)(y2, src_r, dst_r, zeros_nh)
    h2 = _h2_call(y2, dinv, agg2p, b2r, nc)
    return _sim_call(h2)
